# baseline (speedup 1.0000x reference)
import jax
import jax.numpy as jnp
from jax import lax
from jax.experimental import pallas as pl
from jax.experimental.pallas import tpu as pltpu, tpu_sc as plsc

EMBED = 64
BATCH = 16384
NC, NS, L = 2, 16, 16
NW = NC * NS
B_PER_W = BATCH // NW
CH = 64
NCHUNK = B_PER_W // CH


def _body(idx_hbm, ent, rel, o_pos, o_neg, o_ph, o_pt, o_nh, o_nt,
          idxv, b0, b1, b2, b3, b4, b5, c0, c1, c2, c3, c4, c5,
          sem_g, sem_o):
    wid = lax.axis_index("s") * NC + lax.axis_index("c")
    pltpu.sync_copy(idx_hbm.at[:, pl.ds(wid * B_PER_W, B_PER_W)], idxv)

    sets = ((b0, b1, b2, b3, b4, b5), (c0, c1, c2, c3, c4, c5))
    tables = (ent, ent, rel, ent, ent, rel)
    outs = (o_ph, o_pt, o_pos, o_nh, o_nt, o_neg)

    def issue_gathers(c):
        bufs = sets[c % 2]
        return [
            pltpu.async_copy(tables[k].at[idxv.at[k, pl.ds(c * CH, CH)]],
                             bufs[k], sem_g)
            for k in range(6)
        ]

    gh = {0: issue_gathers(0)}
    oh = {}
    for c in range(NCHUNK):
        bufs = sets[c % 2]
        b_ph, b_pt, b_pr, b_nh, b_nt, b_nr = bufs
        for g in gh.pop(c):
            g.wait()
        if c - 1 in oh:
            for o in oh.pop(c - 1):
                o.wait()
        if c + 1 < NCHUNK:
            gh[c + 1] = issue_gathers(c + 1)

        def compute(i, _):
            for j in range(EMBED // L):
                sl = pl.ds(j * L, L)
                b_pr[i, sl] = b_ph[i, sl] + b_pr[i, sl] - b_pt[i, sl]
                b_nr[i, sl] = b_nh[i, sl] + b_nr[i, sl] - b_nt[i, sl]
            return ()

        lax.fori_loop(0, CH, compute, ())

        row0 = wid * B_PER_W + c * CH
        oh[c] = [
            pltpu.async_copy(bufs[k], outs[k].at[pl.ds(row0, CH)], sem_o)
            for k in range(6)
        ]
    for hs in oh.values():
        for o in hs:
            o.wait()


def kernel(pos_h, pos_t, pos_r, neg_h, neg_t, neg_r, ent_emb, rel_emb):
    idx_all = jnp.stack([x.astype(jnp.int32) for x in
                         (pos_h, pos_t, pos_r, neg_h, neg_t, neg_r)])

    out = jax.ShapeDtypeStruct((BATCH, EMBED), jnp.float32)
    run = pl.kernel(
        _body,
        out_type=(out,) * 6,
        mesh=plsc.VectorSubcoreMesh(core_axis_name="c", subcore_axis_name="s"),
        scratch_types=[
            pltpu.VMEM((6, B_PER_W), jnp.int32),
        ] + [pltpu.VMEM((CH, EMBED), jnp.float32)] * 12 + [
            pltpu.SemaphoreType.DMA,
            pltpu.SemaphoreType.DMA,
        ],
        compiler_params=pltpu.CompilerParams(use_tc_tiling_on_sc=False),
    )
    return run(idx_all, ent_emb, rel_emb)
